# Initial kernel scaffold; baseline (speedup 1.0000x reference)
import functools

import jax
import jax.numpy as jnp
from jax import lax
from jax.experimental import pallas as pl
from jax.experimental.pallas import tpu as pltpu
from jax.experimental.pallas import tpu_sc as plsc

N = 10000
E = 640000
G = 64
NC, NS = 2, 16
NT = NC * NS
NROW = 10240
RPT = NROW // NS
DH = 64
K = 128
NCH = 160
EPT = K * NCH

_mesh = plsc.VectorSubcoreMesh(core_axis_name="c", subcore_axis_name="s")


@functools.partial(
    pl.kernel,
    out_type=jax.ShapeDtypeStruct((NC, NROW, 8), jnp.float32),
    mesh=_mesh,
    scratch_types=[
        pltpu.VMEM((NCH, K), jnp.int32),
        pltpu.VMEM((K, 8), jnp.float32),
        pltpu.SemaphoreType.DMA,
    ],
)
def _deg_kernel(dstp_h, zeros_h, ones_h, out_h, dst_v, ones_v, sem):
    c = lax.axis_index("c")
    s = lax.axis_index("s")
    wid = c * NS + s
    row0 = s * RPT

    def scoped(acc_sh):
        pltpu.sync_copy(zeros_h.at[pl.ds(row0, RPT)], acc_sh.at[pl.ds(row0, RPT)])
        pltpu.sync_copy(dstp_h.at[wid], dst_v)
        pltpu.sync_copy(ones_h, ones_v)
        plsc.subcore_barrier()

        def body(q, carry):
            cps = [
                pltpu.async_copy(
                    ones_v, acc_sh.at[dst_v.at[4 * q + u]], sem, add=True
                )
                for u in range(4)
            ]
            for cp in cps:
                cp.wait()
            return carry

        lax.fori_loop(0, NCH // 4, body, 0)
        plsc.subcore_barrier()
        pltpu.sync_copy(
            acc_sh.at[pl.ds(row0, RPT)], out_h.at[c].at[pl.ds(row0, RPT)]
        )

    pl.run_scoped(scoped, pltpu.VMEM_SHARED((NROW, 8), jnp.float32))


@functools.partial(
    pl.kernel,
    out_type=jax.ShapeDtypeStruct((NC, NROW, DH), jnp.float32),
    mesh=_mesh,
    scratch_types=[
        pltpu.VMEM((NCH, K), jnp.int32),
        pltpu.VMEM((NCH, K), jnp.int32),
        pltpu.VMEM((2, K, DH), jnp.float32),
        pltpu.SemaphoreType.DMA,
    ],
)
def _conv_kernel(table_h, srcp_h, dstp_h, zeros_h, out_h, src_v, dst_v, rows_v, gsem):
    c = lax.axis_index("c")
    s = lax.axis_index("s")
    wid = c * NS + s
    row0 = s * RPT

    def scoped(acc_sh):
        pltpu.sync_copy(zeros_h.at[pl.ds(row0, RPT)], acc_sh.at[pl.ds(row0, RPT)])
        pltpu.sync_copy(srcp_h.at[wid], src_v)
        pltpu.sync_copy(dstp_h.at[wid], dst_v)
        plsc.subcore_barrier()

        def g_start(j, slot):
            pltpu.make_async_copy(
                table_h.at[src_v.at[j]], rows_v.at[slot], gsem
            ).start()

        def g_wait(j, slot):
            pltpu.make_async_copy(
                table_h.at[src_v.at[j]], rows_v.at[slot], gsem
            ).wait()

        def s_add(j, slot):
            pltpu.sync_copy(rows_v.at[slot], acc_sh.at[dst_v.at[j]], add=True)

        g_start(0, 0)

        def body(p, carry):
            j0 = 2 * p
            g_wait(j0, 0)
            g_start(j0 + 1, 1)
            s_add(j0, 0)
            g_wait(j0 + 1, 1)

            @pl.when(j0 + 2 < NCH)
            def _():
                g_start(j0 + 2, 0)

            s_add(j0 + 1, 1)
            return carry

        lax.fori_loop(0, NCH // 2, body, 0)
        plsc.subcore_barrier()
        pltpu.sync_copy(
            acc_sh.at[pl.ds(row0, RPT)], out_h.at[c].at[pl.ds(row0, RPT)]
        )

    pl.run_scoped(scoped, pltpu.VMEM_SHARED((NROW, DH), jnp.float32))


def _mm1_body(x_ref, w1_ref, b1_ref, wc1_ref, degp_ref, t1p_ref, dinv_ref):
    h0 = jnp.maximum(
        jnp.dot(x_ref[...], w1_ref[...], preferred_element_type=jnp.float32,
                precision=lax.Precision.HIGHEST) + b1_ref[...],
        0.0,
    )
    d = degp_ref[...]
    deg = d[0, :N, 0] + d[1, :N, 0] + 1.0
    dinv = lax.rsqrt(deg)[:, None]
    t1 = jnp.dot(h0, wc1_ref[...], preferred_element_type=jnp.float32,
                 precision=lax.Precision.HIGHEST)
    t1p_ref[...] = t1 * dinv
    dinv_ref[...] = dinv


def _mm1(x, w1, b1, wc1, degp):
    return pl.pallas_call(
        _mm1_body,
        out_shape=(
            jax.ShapeDtypeStruct((N, DH), jnp.float32),
            jax.ShapeDtypeStruct((N, 1), jnp.float32),
        ),
    )(x, w1, b1, wc1, degp)


def _mm2_body(acc_ref, tp_ref, dinv_ref, bc_ref, wc_ref, out_ref):
    a = acc_ref[...]
    dinv = dinv_ref[...]
    agg = (a[0, :N] + a[1, :N] + tp_ref[...]) * dinv
    h = jnp.maximum(agg + bc_ref[...], 0.0)
    out_ref[...] = jnp.dot(h, wc_ref[...], preferred_element_type=jnp.float32,
                           precision=lax.Precision.HIGHEST) * dinv


def _mm2(acc, tp, dinv, bc, wc):
    return pl.pallas_call(
        _mm2_body,
        out_shape=jax.ShapeDtypeStruct((N, DH), jnp.float32),
    )(acc, tp, dinv, bc, wc)


def _final_body(acc_ref, tp_ref, dinv_ref, bc_ref, batch_ref, w2_ref, b2_ref,
                w3_ref, b3_ref, out_ref):
    a = acc_ref[...]
    agg = (a[0, :N] + a[1, :N] + tp_ref[...]) * dinv_ref[...]
    h2 = jnp.maximum(agg + bc_ref[...], 0.0)
    gids = lax.broadcasted_iota(jnp.int32, (N, G), 1)
    mask = (batch_ref[...] == gids).astype(jnp.float32)
    sums = lax.dot_general(mask, h2, (((0,), (0,)), ((), ())),
                           preferred_element_type=jnp.float32,
                           precision=lax.Precision.HIGHEST)
    cnts = jnp.sum(mask, axis=0)[:, None]
    pooled = sums / jnp.maximum(cnts, 1.0)
    p = jnp.maximum(
        jnp.dot(pooled, w2_ref[...], preferred_element_type=jnp.float32,
                precision=lax.Precision.HIGHEST) + b2_ref[...],
        0.0,
    )
    out_ref[...] = jnp.dot(p, w3_ref[...], preferred_element_type=jnp.float32,
                           precision=lax.Precision.HIGHEST) + b3_ref[...]


def _final(acc, tp, dinv, bc, batch, w2, b2, w3, b3):
    return pl.pallas_call(
        _final_body,
        out_shape=jax.ShapeDtypeStruct((G, 1), jnp.float32),
    )(acc, tp, dinv, bc, batch, w2, b2, w3, b3)


def kernel(x, edge_index, batch, W1, b1, Wc1, bc1, Wc2, bc2, W2, b2, W3, b3):
    pad = EPT - E // NT
    src = edge_index[0].reshape(NT, E // NT)
    dst = edge_index[1].reshape(NT, E // NT)
    pad_src = jnp.zeros((NT, pad), jnp.int32)
    pad_dst = jnp.broadcast_to(
        N + (jnp.arange(pad, dtype=jnp.int32) % (NROW - N)), (NT, pad)
    )
    srcp = jnp.concatenate([src, pad_src], axis=1).reshape(NT, NCH, K)
    dstp = jnp.concatenate([dst, pad_dst], axis=1).reshape(NT, NCH, K)

    zeros8 = jnp.zeros((NROW, 8), jnp.float32)
    ones8 = jnp.ones((K, 8), jnp.float32)
    zeros64 = jnp.zeros((NROW, DH), jnp.float32)

    degp = _deg_kernel(dstp, zeros8, ones8)
    t1p, dinv = _mm1(x, W1, b1.reshape(1, -1), Wc1, degp)
    acc1 = _conv_kernel(t1p, srcp, dstp, zeros64)
    t2p = _mm2(acc1, t1p, dinv, bc1.reshape(1, -1), Wc2)
    acc2 = _conv_kernel(t2p, srcp, dstp, zeros64)
    return _final(acc2, t2p, dinv, bc2.reshape(1, -1), batch.reshape(N, 1),
                  W2, b2.reshape(1, -1), W3, b3.reshape(1, -1))


# speedup vs baseline: 18.2043x; 18.2043x over previous
import functools

import jax
import jax.numpy as jnp
from jax import lax
from jax.experimental import pallas as pl
from jax.experimental.pallas import tpu as pltpu
from jax.experimental.pallas import tpu_sc as plsc

N = 10000
E = 640000
G = 64
NC, NS = 2, 16
NT = NC * NS
NROW = 10240
RPT = NROW // NS
DH = 64
K = 128
NCH = 160
EPT = K * NCH

_mesh = plsc.VectorSubcoreMesh(core_axis_name="c", subcore_axis_name="s")


@functools.partial(
    pl.kernel,
    out_type=jax.ShapeDtypeStruct((NC, NROW, 8), jnp.float32),
    mesh=_mesh,
    scratch_types=[
        pltpu.VMEM((NCH, K), jnp.int32),
        pltpu.VMEM((K, 8), jnp.float32),
        pltpu.VMEM_SHARED((NROW, 8), jnp.float32),
        pltpu.SemaphoreType.DMA,
    ],
)
def _deg_kernel(dstp_h, zeros_h, ones_h, out_h, dst_v, ones_v, acc_sh, sem):
    c = lax.axis_index("c")
    s = lax.axis_index("s")
    wid = c * NS + s
    row0 = s * RPT

    pltpu.sync_copy(zeros_h.at[pl.ds(row0, RPT)], acc_sh.at[pl.ds(row0, RPT)])
    pltpu.sync_copy(dstp_h.at[wid], dst_v)
    pltpu.sync_copy(ones_h, ones_v)
    plsc.subcore_barrier()

    def body(q, carry):
        cps = [
            pltpu.async_copy(
                ones_v, acc_sh.at[dst_v.at[4 * q + u]], sem, add=True
            )
            for u in range(4)
        ]
        for cp in cps:
            cp.wait()
        return carry

    lax.fori_loop(0, NCH // 4, body, 0)
    plsc.subcore_barrier()
    pltpu.sync_copy(
        acc_sh.at[pl.ds(row0, RPT)], out_h.at[c].at[pl.ds(row0, RPT)]
    )


@functools.partial(
    pl.kernel,
    out_type=jax.ShapeDtypeStruct((NC, NROW, DH), jnp.float32),
    mesh=_mesh,
    scratch_types=[
        pltpu.VMEM((NCH, K), jnp.int32),
        pltpu.VMEM((NCH, K), jnp.int32),
        pltpu.VMEM((2, K, DH), jnp.float32),
        pltpu.VMEM_SHARED((NROW, DH), jnp.float32),
        pltpu.SemaphoreType.DMA,
    ],
    compiler_params=pltpu.CompilerParams(use_tc_tiling_on_sc=False),
)
def _conv_kernel(table_h, srcp_h, dstp_h, zeros_h, out_h, src_v, dst_v, rows_v,
                 acc_sh, gsem):
    c = lax.axis_index("c")
    s = lax.axis_index("s")
    wid = c * NS + s
    row0 = s * RPT

    pltpu.sync_copy(zeros_h.at[pl.ds(row0, RPT)], acc_sh.at[pl.ds(row0, RPT)])
    pltpu.sync_copy(srcp_h.at[wid], src_v)
    pltpu.sync_copy(dstp_h.at[wid], dst_v)
    plsc.subcore_barrier()

    def g_start(j, slot):
        pltpu.make_async_copy(
            table_h.at[src_v.at[j]], rows_v.at[slot], gsem
        ).start()

    def g_wait(j, slot):
        pltpu.make_async_copy(
            table_h.at[src_v.at[j]], rows_v.at[slot], gsem
        ).wait()

    def s_add(j, slot):
        pltpu.sync_copy(rows_v.at[slot], acc_sh.at[dst_v.at[j]], add=True)

    g_start(0, 0)

    def body(p, carry):
        j0 = 2 * p
        g_wait(j0, 0)
        g_start(j0 + 1, 1)
        s_add(j0, 0)
        g_wait(j0 + 1, 1)

        @pl.when(j0 + 2 < NCH)
        def _():
            g_start(j0 + 2, 0)

        s_add(j0 + 1, 1)
        return carry

    lax.fori_loop(0, NCH // 2, body, 0)
    plsc.subcore_barrier()
    pltpu.sync_copy(
        acc_sh.at[pl.ds(row0, RPT)], out_h.at[c].at[pl.ds(row0, RPT)]
    )


def _mm1_body(x_ref, w1_ref, b1_ref, wc1_ref, degp_ref, t1p_ref, dinv_ref):
    h0 = jnp.maximum(
        jnp.dot(x_ref[...], w1_ref[...], preferred_element_type=jnp.float32,
                precision=lax.Precision.HIGHEST) + b1_ref[...],
        0.0,
    )
    d = degp_ref[...]
    deg = d[0, :N, 0] + d[1, :N, 0] + 1.0
    dinv = lax.rsqrt(deg)[:, None]
    t1 = jnp.dot(h0, wc1_ref[...], preferred_element_type=jnp.float32,
                 precision=lax.Precision.HIGHEST)
    t1p_ref[...] = t1 * dinv
    dinv_ref[...] = dinv


def _mm1(x, w1, b1, wc1, degp):
    return pl.pallas_call(
        _mm1_body,
        out_shape=(
            jax.ShapeDtypeStruct((N, DH), jnp.float32),
            jax.ShapeDtypeStruct((N, 1), jnp.float32),
        ),
    )(x, w1, b1, wc1, degp)


def _mm2_body(acc_ref, tp_ref, dinv_ref, bc_ref, wc_ref, out_ref):
    a = acc_ref[...]
    dinv = dinv_ref[...]
    agg = (a[0, :N] + a[1, :N] + tp_ref[...]) * dinv
    h = jnp.maximum(agg + bc_ref[...], 0.0)
    out_ref[...] = jnp.dot(h, wc_ref[...], preferred_element_type=jnp.float32,
                           precision=lax.Precision.HIGHEST) * dinv


def _mm2(acc, tp, dinv, bc, wc):
    return pl.pallas_call(
        _mm2_body,
        out_shape=jax.ShapeDtypeStruct((N, DH), jnp.float32),
    )(acc, tp, dinv, bc, wc)


def _final_body(acc_ref, tp_ref, dinv_ref, bc_ref, batch_ref, w2_ref, b2_ref,
                w3_ref, b3_ref, out_ref):
    a = acc_ref[...]
    agg = (a[0, :N] + a[1, :N] + tp_ref[...]) * dinv_ref[...]
    h2 = jnp.maximum(agg + bc_ref[...], 0.0)
    gids = lax.broadcasted_iota(jnp.int32, (N, G), 1)
    mask = (batch_ref[...] == gids).astype(jnp.float32)
    sums = lax.dot_general(mask, h2, (((0,), (0,)), ((), ())),
                           preferred_element_type=jnp.float32,
                           precision=lax.Precision.HIGHEST)
    cnts = jnp.sum(mask, axis=0)[:, None]
    pooled = sums / jnp.maximum(cnts, 1.0)
    p = jnp.maximum(
        jnp.dot(pooled, w2_ref[...], preferred_element_type=jnp.float32,
                precision=lax.Precision.HIGHEST) + b2_ref[...],
        0.0,
    )
    out_ref[...] = jnp.dot(p, w3_ref[...], preferred_element_type=jnp.float32,
                           precision=lax.Precision.HIGHEST) + b3_ref[...]


def _final(acc, tp, dinv, bc, batch, w2, b2, w3, b3):
    return pl.pallas_call(
        _final_body,
        out_shape=jax.ShapeDtypeStruct((G, 1), jnp.float32),
    )(acc, tp, dinv, bc, batch, w2, b2, w3, b3)


def kernel(x, edge_index, batch, W1, b1, Wc1, bc1, Wc2, bc2, W2, b2, W3, b3):
    pad = EPT - E // NT
    src = edge_index[0].reshape(NT, E // NT)
    dst = edge_index[1].reshape(NT, E // NT)
    pad_src = jnp.zeros((NT, pad), jnp.int32)
    pad_dst = jnp.broadcast_to(
        N + (jnp.arange(pad, dtype=jnp.int32) % (NROW - N)), (NT, pad)
    )
    srcp = jnp.concatenate([src, pad_src], axis=1).reshape(NT, NCH, K)
    dstp = jnp.concatenate([dst, pad_dst], axis=1).reshape(NT, NCH, K)

    zeros8 = jnp.zeros((NROW, 8), jnp.float32)
    ones8 = jnp.ones((K, 8), jnp.float32)
    zeros64 = jnp.zeros((NROW, DH), jnp.float32)

    degp = _deg_kernel(dstp, zeros8, ones8)
    t1p, dinv = _mm1(x, W1, b1.reshape(1, -1), Wc1, degp)
    acc1 = _conv_kernel(t1p, srcp, dstp, zeros64)
    t2p = _mm2(acc1, t1p, dinv, bc1.reshape(1, -1), Wc2)
    acc2 = _conv_kernel(t2p, srcp, dstp, zeros64)
    return _final(acc2, t2p, dinv, bc2.reshape(1, -1), batch.reshape(N, 1),
                  W2, b2.reshape(1, -1), W3, b3.reshape(1, -1))


# speedup vs baseline: 20.9207x; 1.1492x over previous
import functools

import jax
import jax.numpy as jnp
from jax import lax
from jax.experimental import pallas as pl
from jax.experimental.pallas import tpu as pltpu
from jax.experimental.pallas import tpu_sc as plsc

N = 10000
E = 640000
G = 64
NC, NS = 2, 16
NT = NC * NS
NROW = 10240
RPT = NROW // NS
DH = 64
K = 128
NCH = 160
EPT = K * NCH

_mesh = plsc.VectorSubcoreMesh(core_axis_name="c", subcore_axis_name="s")


@functools.partial(
    pl.kernel,
    out_type=jax.ShapeDtypeStruct((NC, NROW, 8), jnp.float32),
    mesh=_mesh,
    scratch_types=[
        pltpu.VMEM((NCH, K), jnp.int32),
        pltpu.VMEM((K, 8), jnp.float32),
        pltpu.VMEM_SHARED((NROW, 8), jnp.float32),
        pltpu.SemaphoreType.DMA,
    ],
)
def _deg_kernel(dstp_h, zeros_h, ones_h, out_h, dst_v, ones_v, acc_sh, sem):
    c = lax.axis_index("c")
    s = lax.axis_index("s")
    wid = c * NS + s
    row0 = s * RPT

    pltpu.sync_copy(zeros_h.at[pl.ds(row0, RPT)], acc_sh.at[pl.ds(row0, RPT)])
    pltpu.sync_copy(dstp_h.at[wid], dst_v)
    pltpu.sync_copy(ones_h, ones_v)
    plsc.subcore_barrier()

    def body(q, carry):
        cps = [
            pltpu.async_copy(
                ones_v, acc_sh.at[dst_v.at[4 * q + u]], sem, add=True
            )
            for u in range(4)
        ]
        for cp in cps:
            cp.wait()
        return carry

    lax.fori_loop(0, NCH // 4, body, 0)
    plsc.subcore_barrier()
    pltpu.sync_copy(
        acc_sh.at[pl.ds(row0, RPT)], out_h.at[c].at[pl.ds(row0, RPT)]
    )


@functools.partial(
    pl.kernel,
    out_type=jax.ShapeDtypeStruct((NC, NROW, DH), jnp.float32),
    mesh=_mesh,
    scratch_types=[
        pltpu.VMEM((NCH, K), jnp.int32),
        pltpu.VMEM((NCH, K), jnp.int32),
        pltpu.VMEM((4, K, DH), jnp.float32),
        pltpu.VMEM_SHARED((NROW, DH), jnp.float32),
        pltpu.SemaphoreType.DMA,
        pltpu.SemaphoreType.DMA,
    ],
    compiler_params=pltpu.CompilerParams(use_tc_tiling_on_sc=False),
)
def _conv_kernel(table_h, srcp_h, dstp_h, zeros_h, out_h, src_v, dst_v, rows_v,
                 acc_sh, gsem, ssem):
    c = lax.axis_index("c")
    s = lax.axis_index("s")
    wid = c * NS + s
    row0 = s * RPT

    pltpu.sync_copy(zeros_h.at[pl.ds(row0, RPT)], acc_sh.at[pl.ds(row0, RPT)])
    pltpu.sync_copy(srcp_h.at[wid], src_v)
    pltpu.sync_copy(dstp_h.at[wid], dst_v)
    plsc.subcore_barrier()

    def g_start(j, slot):
        pltpu.make_async_copy(
            table_h.at[src_v.at[j]], rows_v.at[slot], gsem
        ).start()

    def g_wait(j, slot):
        pltpu.make_async_copy(
            table_h.at[src_v.at[j]], rows_v.at[slot], gsem
        ).wait()

    def s_start(j, slot):
        pltpu.async_copy(rows_v.at[slot], acc_sh.at[dst_v.at[j]], ssem, add=True)

    def s_wait(j, slot):
        pltpu.make_async_copy(
            rows_v.at[slot], acc_sh.at[dst_v.at[j]], ssem
        ).wait()

    g_start(0, 0)
    g_start(1, 1)
    g_start(2, 2)

    def body(p, carry):
        for u in range(4):
            j = 4 * p + u
            g_wait(j, u)
            s_start(j, u)

            @pl.when(j >= 1)
            def _():
                s_wait(j - 1, (u + 3) % 4)

            @pl.when(j + 3 < NCH)
            def _():
                g_start(j + 3, (u + 3) % 4)

        return carry

    lax.fori_loop(0, NCH // 4, body, 0)
    s_wait(NCH - 1, 3)
    plsc.subcore_barrier()
    pltpu.sync_copy(
        acc_sh.at[pl.ds(row0, RPT)], out_h.at[c].at[pl.ds(row0, RPT)]
    )


def _mm1_body(x_ref, w1_ref, b1_ref, wc1_ref, degp_ref, t1p_ref, dinv_ref):
    h0 = jnp.maximum(
        jnp.dot(x_ref[...], w1_ref[...], preferred_element_type=jnp.float32,
                precision=lax.Precision.HIGHEST) + b1_ref[...],
        0.0,
    )
    d = degp_ref[...]
    deg = d[0, :N, 0] + d[1, :N, 0] + 1.0
    dinv = lax.rsqrt(deg)[:, None]
    t1 = jnp.dot(h0, wc1_ref[...], preferred_element_type=jnp.float32,
                 precision=lax.Precision.HIGHEST)
    t1p_ref[...] = t1 * dinv
    dinv_ref[...] = dinv


def _mm1(x, w1, b1, wc1, degp):
    return pl.pallas_call(
        _mm1_body,
        out_shape=(
            jax.ShapeDtypeStruct((N, DH), jnp.float32),
            jax.ShapeDtypeStruct((N, 1), jnp.float32),
        ),
    )(x, w1, b1, wc1, degp)


def _mm2_body(acc_ref, tp_ref, dinv_ref, bc_ref, wc_ref, out_ref):
    a = acc_ref[...]
    dinv = dinv_ref[...]
    agg = (a[0, :N] + a[1, :N] + tp_ref[...]) * dinv
    h = jnp.maximum(agg + bc_ref[...], 0.0)
    out_ref[...] = jnp.dot(h, wc_ref[...], preferred_element_type=jnp.float32,
                           precision=lax.Precision.HIGHEST) * dinv


def _mm2(acc, tp, dinv, bc, wc):
    return pl.pallas_call(
        _mm2_body,
        out_shape=jax.ShapeDtypeStruct((N, DH), jnp.float32),
    )(acc, tp, dinv, bc, wc)


def _final_body(acc_ref, tp_ref, dinv_ref, bc_ref, batch_ref, w2_ref, b2_ref,
                w3_ref, b3_ref, out_ref):
    a = acc_ref[...]
    agg = (a[0, :N] + a[1, :N] + tp_ref[...]) * dinv_ref[...]
    h2 = jnp.maximum(agg + bc_ref[...], 0.0)
    gids = lax.broadcasted_iota(jnp.int32, (N, G), 1)
    mask = (batch_ref[...] == gids).astype(jnp.float32)
    sums = lax.dot_general(mask, h2, (((0,), (0,)), ((), ())),
                           preferred_element_type=jnp.float32,
                           precision=lax.Precision.HIGHEST)
    cnts = jnp.sum(mask, axis=0)[:, None]
    pooled = sums / jnp.maximum(cnts, 1.0)
    p = jnp.maximum(
        jnp.dot(pooled, w2_ref[...], preferred_element_type=jnp.float32,
                precision=lax.Precision.HIGHEST) + b2_ref[...],
        0.0,
    )
    out_ref[...] = jnp.dot(p, w3_ref[...], preferred_element_type=jnp.float32,
                           precision=lax.Precision.HIGHEST) + b3_ref[...]


def _final(acc, tp, dinv, bc, batch, w2, b2, w3, b3):
    return pl.pallas_call(
        _final_body,
        out_shape=jax.ShapeDtypeStruct((G, 1), jnp.float32),
    )(acc, tp, dinv, bc, batch, w2, b2, w3, b3)


def kernel(x, edge_index, batch, W1, b1, Wc1, bc1, Wc2, bc2, W2, b2, W3, b3):
    pad = EPT - E // NT
    src = edge_index[0].reshape(NT, E // NT)
    dst = edge_index[1].reshape(NT, E // NT)
    pad_src = jnp.zeros((NT, pad), jnp.int32)
    pad_dst = jnp.broadcast_to(
        N + (jnp.arange(pad, dtype=jnp.int32) % (NROW - N)), (NT, pad)
    )
    srcp = jnp.concatenate([src, pad_src], axis=1).reshape(NT, NCH, K)
    dstp = jnp.concatenate([dst, pad_dst], axis=1).reshape(NT, NCH, K)

    zeros8 = jnp.zeros((NROW, 8), jnp.float32)
    ones8 = jnp.ones((K, 8), jnp.float32)
    zeros64 = jnp.zeros((NROW, DH), jnp.float32)

    degp = _deg_kernel(dstp, zeros8, ones8)
    t1p, dinv = _mm1(x, W1, b1.reshape(1, -1), Wc1, degp)
    acc1 = _conv_kernel(t1p, srcp, dstp, zeros64)
    t2p = _mm2(acc1, t1p, dinv, bc1.reshape(1, -1), Wc2)
    acc2 = _conv_kernel(t2p, srcp, dstp, zeros64)
    return _final(acc2, t2p, dinv, bc2.reshape(1, -1), batch.reshape(N, 1),
                  W2, b2.reshape(1, -1), W3, b3.reshape(1, -1))
